# baseline (speedup 1.0000x reference)
import functools

import jax
import jax.numpy as jnp
from jax import lax
from jax.experimental import pallas as pl
from jax.experimental.pallas import tpu as pltpu
from jax.experimental.pallas import tpu_sc as plsc

_TR = 2048
_SEG = 8
_TAIL = 512


def _repack_body(t_ref, tail_ref, o_ref, seg, sems):
    i = pl.program_id(0)
    nb = pl.num_programs(0)
    D = t_ref.shape[0]

    def issue(block, slot):
        base = block * (_SEG * _TR)
        for s in range(_SEG):
            pltpu.make_async_copy(
                t_ref.at[:, pl.ds(base + s * _TR, _TR)],
                seg.at[slot, pl.ds(D * s, D)], sems.at[slot, s]).start()

    def issue_tail(slot):
        pltpu.make_async_copy(
            t_ref.at[:, pl.ds((nb - 1) * _SEG * _TR, _TAIL)],
            seg.at[slot, pl.ds(0, D), pl.ds(0, _TAIL)], sems.at[slot, 0]).start()

    @pl.when(i == 0)
    def _():
        issue(0, 0)

    buf = lax.rem(i, 2)

    @pl.when(i + 1 < nb - 1)
    def _():
        issue(i + 1, 1 - buf)

    @pl.when(i + 1 == nb - 1)
    def _():
        issue_tail(1 - buf)

    @pl.when(i < nb - 1)
    def _():
        for s in range(_SEG):
            pltpu.make_async_copy(
                t_ref.at[:, pl.ds(0, _TR)], seg.at[buf, pl.ds(D * s, D)],
                sems.at[buf, s]).wait()

    @pl.when(i == nb - 1)
    def _():
        pltpu.make_async_copy(
            t_ref.at[:, pl.ds(0, _TAIL)],
            seg.at[buf, pl.ds(0, D), pl.ds(0, _TAIL)], sems.at[buf, 0]).wait()

    packed = pltpu.bitcast(seg[buf].astype(jnp.bfloat16), jnp.float32)
    o_ref[...] = jnp.transpose(packed, (1, 0))

    @pl.when(i == nb - 1)
    def _():
        nt = tail_ref.shape[0]
        o_ref[_TR - nt:, :] = tail_ref[...]


def _repack(table_t, tail):
    D, V = table_t.shape
    NB = pl.cdiv(V, _SEG * _TR)
    W = _SEG * D // 2
    return pl.pallas_call(
        _repack_body,
        grid=(NB,),
        in_specs=[pl.BlockSpec(memory_space=pl.ANY),
                  pl.BlockSpec((tail.shape[0], W), lambda i: (0, 0))],
        out_specs=pl.BlockSpec((_TR, W), lambda i: (i, 0)),
        out_shape=jax.ShapeDtypeStruct((NB * _TR, W), jnp.float32),
        scratch_shapes=[
            pltpu.VMEM((2, _SEG * D, _TR), jnp.float32),
            pltpu.SemaphoreType.DMA((2, _SEG)),
        ],
    )(table_t, tail)


def _make_pool_kernel(B, D, n_ids):
    info = plsc.get_sparse_core_info()
    NC, NS, L = info.num_cores, info.num_subcores, info.num_lanes
    NW = NC * NS
    assert D == 2 * L and n_ids == 10
    assert B % NW == 0
    b_per_w = B // NW
    CHUNK = 128
    assert b_per_w % CHUNK == 0
    n_chunks = b_per_w // CHUNK
    ids_per_chunk = CHUNK * n_ids
    assert ids_per_chunk % 128 == 0
    idx_rows = ids_per_chunk // 128
    ids_w = b_per_w * n_ids
    assert ids_w % 8 == 0

    mesh = plsc.VectorSubcoreMesh(core_axis_name="c", subcore_axis_name="s")

    @functools.partial(
        pl.kernel,
        mesh=mesh,
        compiler_params=pltpu.CompilerParams(
            use_tc_tiling_on_sc=False, needs_layout_passes=False),
        out_type=jax.ShapeDtypeStruct((B, 2 * D), jnp.float32),
        scratch_types=[
            pltpu.VMEM((ids_w,), jnp.int32),
            pltpu.VMEM((ids_per_chunk, D // 2), jnp.float32),
            pltpu.VMEM((CHUNK, 2 * D), jnp.float32),
            pltpu.SemaphoreType.DMA,
        ],
    )
    def pool_kernel(ids_hbm, table_hbm, out_hbm, idx_v, rows_v, pool_v, sem):
        wid = lax.axis_index("s") * NC + lax.axis_index("c")
        pltpu.sync_copy(ids_hbm.at[pl.ds(wid * ids_w, ids_w)], idx_v)

        for c in range(n_chunks):
            handles = []
            for j in range(idx_rows):
                handles.append(
                    pltpu.async_copy(
                        table_hbm.at[
                            idx_v.at[pl.ds(c * ids_per_chunk + j * 128, 128)]],
                        rows_v.at[pl.ds(j * 128, 128)],
                        sem,
                    )
                )
            for h in handles:
                h.wait()

            def body(i, carry):
                i10 = i * n_ids
                for t in range(2):
                    acc_a = None
                    for j in range(5):
                        wv = rows_v[i10 + 5 * t + j, pl.ds(0, L)]
                        a, b = plsc.unpack(plsc.bitcast(wv, jnp.bfloat16),
                                           format=plsc.PackFormat.INTERLEAVED)
                        if acc_a is None:
                            acc_a, acc_b = a, b
                        else:
                            acc_a = acc_a + a
                            acc_b = acc_b + b
                    pool_v[i, pl.ds(t * D, L)] = acc_a * 0.2
                    pool_v[i, pl.ds(t * D + L, L)] = acc_b * 0.2
                return carry

            lax.fori_loop(0, CHUNK, body, 0)

            out_base = wid * b_per_w + c * CHUNK
            pltpu.sync_copy(pool_v, out_hbm.at[pl.ds(out_base, CHUNK)])

    return pool_kernel


def _mlp_body(p_ref, e_ref, w1a_ref, w1b_ref, b1_ref, w2_ref, b2_ref, o_ref):
    h = jnp.dot(p_ref[...], w1a_ref[...], preferred_element_type=jnp.float32)
    h = h + jnp.dot(e_ref[...], w1b_ref[...], preferred_element_type=jnp.float32)
    h = jnp.maximum(h + b1_ref[...], 0.0)
    o_ref[...] = jnp.sum(h * w2_ref[...], axis=1) + b2_ref[0]


def _mlp(pooled, extras, W1a, W1b, b1r, w2t, b2, Bt=1024):
    B, F = pooled.shape
    E = extras.shape[1]
    H = W1a.shape[1]
    grid = (B // Bt,)
    return pl.pallas_call(
        _mlp_body,
        grid=grid,
        in_specs=[
            pl.BlockSpec((Bt, F), lambda i: (i, 0)),
            pl.BlockSpec((Bt, E), lambda i: (i, 0)),
            pl.BlockSpec((F, H), lambda i: (0, 0)),
            pl.BlockSpec((E, H), lambda i: (0, 0)),
            pl.BlockSpec((1, H), lambda i: (0, 0)),
            pl.BlockSpec((1, H), lambda i: (0, 0)),
            pl.BlockSpec(memory_space=pltpu.SMEM),
        ],
        out_specs=pl.BlockSpec((Bt,), lambda i: (i,)),
        out_shape=jax.ShapeDtypeStruct((B,), jnp.float32),
    )(pooled, extras, W1a, W1b, b1r, w2t, b2)


def kernel(radiant_ids, dire_ids, avg_rank_tiers, num_rank_tiers, durations,
           emb_table, W1, b1, W2, b2):
    B = radiant_ids.shape[0]
    V, D = emb_table.shape
    H = W1.shape[1]

    NB = pl.cdiv(V, _SEG * _TR)
    NR = NB * _TR
    tail_base = (NB - 1) * _SEG * _TR + _TAIL
    n_tail = V - tail_base
    spare_q = NR - n_tail // _SEG
    tailp = lax.bitcast_convert_type(
        emb_table[tail_base:].astype(jnp.bfloat16).reshape(
            n_tail // _SEG, _SEG * D // 2, 2), jnp.float32)
    rp = _repack(emb_table.T, tailp)

    table_lin = rp.reshape(_SEG * NR, D // 2)

    ids = jnp.concatenate(
        [radiant_ids.astype(jnp.int32), dire_ids.astype(jnp.int32)], axis=1)
    w = ids % (_SEG * _TR)
    rows = (ids - w) + _SEG * (w % _TR) + (w // _TR)
    rows = jnp.where(ids >= tail_base,
                     _SEG * spare_q + (ids - tail_base), rows)

    pooled = _make_pool_kernel(B, D, 10)(rows.reshape(B * 10), table_lin)

    extras = jnp.stack([avg_rank_tiers, num_rank_tiers, durations], axis=1)
    perm = jnp.array([t * D + c for t in range(2)
                      for c in list(range(0, D, 2)) + list(range(1, D, 2))])
    W1a = W1[: 2 * D][perm]
    W1b = W1[2 * D:]

    return _mlp(pooled, extras, W1a, W1b,
                b1.reshape(1, H), W2.reshape(1, H), b2)


# speedup vs baseline: 1.0640x; 1.0640x over previous
import functools

import jax
import jax.numpy as jnp
from jax import lax
from jax.experimental import pallas as pl
from jax.experimental.pallas import tpu as pltpu
from jax.experimental.pallas import tpu_sc as plsc

_TR = 2048
_SEG = 8
_TAIL = 512


def _repack_body(t_ref, tail_ref, o_ref, seg, sems):
    i = pl.program_id(0)
    nb = pl.num_programs(0)
    D = t_ref.shape[0]

    def issue(block, slot):
        base = block * (_SEG * _TR)
        for s in range(_SEG):
            pltpu.make_async_copy(
                t_ref.at[:, pl.ds(base + s * _TR, _TR)],
                seg.at[slot, pl.ds(D * s, D)], sems.at[slot, s]).start()

    def issue_tail(slot):
        pltpu.make_async_copy(
            t_ref.at[:, pl.ds((nb - 1) * _SEG * _TR, _TAIL)],
            seg.at[slot, pl.ds(0, D), pl.ds(0, _TAIL)], sems.at[slot, 0]).start()

    @pl.when(i == 0)
    def _():
        issue(0, 0)

    buf = lax.rem(i, 2)

    @pl.when(i + 1 < nb - 1)
    def _():
        issue(i + 1, 1 - buf)

    @pl.when(i + 1 == nb - 1)
    def _():
        issue_tail(1 - buf)

    @pl.when(i < nb - 1)
    def _():
        for s in range(_SEG):
            pltpu.make_async_copy(
                t_ref.at[:, pl.ds(0, _TR)], seg.at[buf, pl.ds(D * s, D)],
                sems.at[buf, s]).wait()

    @pl.when(i == nb - 1)
    def _():
        pltpu.make_async_copy(
            t_ref.at[:, pl.ds(0, _TAIL)],
            seg.at[buf, pl.ds(0, D), pl.ds(0, _TAIL)], sems.at[buf, 0]).wait()

    packed = pltpu.bitcast(seg[buf].astype(jnp.bfloat16), jnp.float32)
    o_ref[...] = jnp.transpose(packed, (1, 0))

    @pl.when(i == nb - 1)
    def _():
        nt = tail_ref.shape[0]
        o_ref[_TR - nt:, :] = tail_ref[...]


def _repack(table_t, tail):
    D, V = table_t.shape
    NB = pl.cdiv(V, _SEG * _TR)
    W = _SEG * D // 2
    return pl.pallas_call(
        _repack_body,
        grid=(NB,),
        in_specs=[pl.BlockSpec(memory_space=pl.ANY),
                  pl.BlockSpec((tail.shape[0], W), lambda i: (0, 0))],
        out_specs=pl.BlockSpec((_TR, W), lambda i: (i, 0)),
        out_shape=jax.ShapeDtypeStruct((NB * _TR, W), jnp.float32),
        scratch_shapes=[
            pltpu.VMEM((2, _SEG * D, _TR), jnp.float32),
            pltpu.SemaphoreType.DMA((2, _SEG)),
        ],
    )(table_t, tail)


def _make_pool_kernel(B, D, n_ids):
    info = plsc.get_sparse_core_info()
    NC, NS, L = info.num_cores, info.num_subcores, info.num_lanes
    NW = NC * NS
    assert D == 2 * L and n_ids == 10
    assert B % NW == 0
    b_per_w = B // NW
    CHUNK = 128
    assert b_per_w % CHUNK == 0
    n_chunks = b_per_w // CHUNK
    ids_per_chunk = CHUNK * n_ids
    assert ids_per_chunk % 128 == 0
    idx_rows = ids_per_chunk // 128
    ids_w = b_per_w * n_ids
    assert ids_w % 8 == 0

    mesh = plsc.VectorSubcoreMesh(core_axis_name="c", subcore_axis_name="s")

    @functools.partial(
        pl.kernel,
        mesh=mesh,
        compiler_params=pltpu.CompilerParams(
            use_tc_tiling_on_sc=False, needs_layout_passes=False),
        out_type=jax.ShapeDtypeStruct((B, 2 * D), jnp.float32),
        scratch_types=[
            pltpu.VMEM((ids_w,), jnp.int32),
            pltpu.VMEM((ids_per_chunk, D // 2), jnp.float32),
            pltpu.VMEM((CHUNK, 2 * D), jnp.float32),
            pltpu.SemaphoreType.DMA,
        ],
    )
    def pool_kernel(ids_hbm, table_hbm, out_hbm, idx_v, rows_v, pool_v, sem):
        wid = lax.axis_index("s") * NC + lax.axis_index("c")
        pltpu.sync_copy(ids_hbm.at[pl.ds(wid * ids_w, ids_w)], idx_v)

        for c in range(n_chunks):
            handles = []
            for j in range(idx_rows):
                handles.append(
                    pltpu.async_copy(
                        table_hbm.at[
                            idx_v.at[pl.ds(c * ids_per_chunk + j * 128, 128)]],
                        rows_v.at[pl.ds(j * 128, 128)],
                        sem,
                    )
                )
            for h in handles:
                h.wait()

            def body(i, carry):
                i10 = i * n_ids
                for t in range(2):
                    acc_a = None
                    for j in range(5):
                        wv = rows_v[i10 + 5 * t + j, pl.ds(0, L)]
                        a, b = plsc.unpack(plsc.bitcast(wv, jnp.bfloat16),
                                           format=plsc.PackFormat.INTERLEAVED)
                        if acc_a is None:
                            acc_a, acc_b = a, b
                        else:
                            acc_a = acc_a + a
                            acc_b = acc_b + b
                    pool_v[i, pl.ds(t * D, L)] = acc_a * 0.2
                    pool_v[i, pl.ds(t * D + L, L)] = acc_b * 0.2
                return carry

            lax.fori_loop(0, CHUNK, body, 0)

            out_base = wid * b_per_w + c * CHUNK
            pltpu.sync_copy(pool_v, out_hbm.at[pl.ds(out_base, CHUNK)])

    return pool_kernel


def _mlp_body(p_ref, e_ref, w1a_ref, w1b_ref, b1_ref, w2_ref, b2_ref, o_ref):
    h = jnp.dot(p_ref[...], w1a_ref[...], preferred_element_type=jnp.float32)
    h = h + jnp.dot(e_ref[...], w1b_ref[...], preferred_element_type=jnp.float32)
    h = jnp.maximum(h + b1_ref[...], 0.0)
    o_ref[...] = jnp.sum(h * w2_ref[...], axis=1, keepdims=True) + b2_ref[0]


def _mlp(pooled, extras, W1a, W1b, b1r, w2t, b2, Bt=1024):
    B, F = pooled.shape
    E = extras.shape[1]
    H = W1a.shape[1]
    grid = (B // Bt,)
    return pl.pallas_call(
        _mlp_body,
        grid=grid,
        in_specs=[
            pl.BlockSpec((Bt, F), lambda i: (i, 0)),
            pl.BlockSpec((Bt, E), lambda i: (i, 0)),
            pl.BlockSpec((F, H), lambda i: (0, 0)),
            pl.BlockSpec((E, H), lambda i: (0, 0)),
            pl.BlockSpec((1, H), lambda i: (0, 0)),
            pl.BlockSpec((1, H), lambda i: (0, 0)),
            pl.BlockSpec(memory_space=pltpu.SMEM),
        ],
        out_specs=pl.BlockSpec((Bt, 1), lambda i: (i, 0)),
        out_shape=jax.ShapeDtypeStruct((B, 1), jnp.float32),
    )(pooled, extras, W1a, W1b, b1r, w2t, b2)


def kernel(radiant_ids, dire_ids, avg_rank_tiers, num_rank_tiers, durations,
           emb_table, W1, b1, W2, b2):
    B = radiant_ids.shape[0]
    V, D = emb_table.shape
    H = W1.shape[1]

    NB = pl.cdiv(V, _SEG * _TR)
    NR = NB * _TR
    tail_base = (NB - 1) * _SEG * _TR + _TAIL
    n_tail = V - tail_base
    spare_q = NR - n_tail // _SEG
    tailp = lax.bitcast_convert_type(
        emb_table[tail_base:].astype(jnp.bfloat16).reshape(
            n_tail // _SEG, _SEG * D // 2, 2), jnp.float32)
    rp = _repack(emb_table.T, tailp)

    table_lin = rp.reshape(_SEG * NR, D // 2)

    ids = jnp.concatenate(
        [radiant_ids.astype(jnp.int32), dire_ids.astype(jnp.int32)], axis=1)
    w = ids % (_SEG * _TR)
    rows = (ids - w) + _SEG * (w % _TR) + (w // _TR)
    rows = jnp.where(ids >= tail_base,
                     _SEG * spare_q + (ids - tail_base), rows)

    pooled = _make_pool_kernel(B, D, 10)(rows.reshape(B * 10), table_lin)

    extras = jnp.stack([avg_rank_tiers, num_rank_tiers, durations], axis=1)
    perm = jnp.array([t * D + c for t in range(2)
                      for c in list(range(0, D, 2)) + list(range(1, D, 2))])
    W1a = W1[: 2 * D][perm]
    W1b = W1[2 * D:]

    logit = _mlp(pooled, extras, W1a, W1b,
                 b1.reshape(1, H), W2.reshape(1, H), b2)
    return logit.reshape(B)
